# Initial kernel scaffold; baseline (speedup 1.0000x reference)
import jax
import jax.numpy as jnp
from jax.experimental import pallas as pl


def kernel(x, half_edges, W0, b0, W1, b1, W2, b2, Wfc, bfc):
    raise NotImplementedError("write your pallas kernel here")


# speedup vs baseline: 7.1083x; 7.1083x over previous
import functools

import jax
import jax.numpy as jnp
from jax import lax
from jax.experimental import pallas as pl
from jax.experimental.pallas import tpu as pltpu
from jax.experimental.pallas import tpu_sc as plsc

N = 320000
C = 128
K = 4
P = 16
CAT = 32

NC = 2
NS = 16
NW = NC * NS
ROWS_W = N // NW
B = 80
NCH = ROWS_W // B
LANES = 16


def _proj_body(x_ref, ws_ref, wn_ref, b_ref, zself_ref, zn_ref):
    xb = x_ref[...]
    zself_ref[...] = (
        jnp.dot(xb, ws_ref[...], preferred_element_type=jnp.float32)
        + b_ref[...]
    )
    for k in range(K):
        zn_ref[k] = jnp.dot(xb, wn_ref[k], preferred_element_type=jnp.float32)


@functools.lru_cache(maxsize=None)
def _make_project(n, bn):
    grid = n // bn
    return pl.pallas_call(
        _proj_body,
        grid=(grid,),
        in_specs=[
            pl.BlockSpec((bn, C), lambda i: (i, 0)),
            pl.BlockSpec((C, C), lambda i: (0, 0)),
            pl.BlockSpec((K, C, C), lambda i: (0, 0, 0)),
            pl.BlockSpec((1, C), lambda i: (0, 0)),
        ],
        out_specs=[
            pl.BlockSpec((bn, C), lambda i: (i, 0)),
            pl.BlockSpec((K, bn, C), lambda i: (0, i, 0)),
        ],
        out_shape=[
            jax.ShapeDtypeStruct((n, C), jnp.float32),
            jax.ShapeDtypeStruct((K, n, C), jnp.float32),
        ],
    )


def _sc_combine_rows(acc_v, gbuf_v, pool_v):
    def row(r, carry):
        for cc in range(C // LANES):
            sl = pl.ds(cc * LANES, LANES)
            v = acc_v[r, sl]
            for k in range(K):
                v = v + gbuf_v[k, r, sl]
            v = jnp.maximum(v, 0.0)
            if pool_v is None:
                acc_v[r, sl] = v
            else:
                pool_v[sl] = pool_v[sl] + v
        return carry

    lax.fori_loop(0, B, row, 0)


def _sc_body(pool, zself_hbm, znf_hbm, idx_hbm, out_hbm,
             idx_v, acc_v, gbuf_v, pool_v, sem):
    wid = lax.axis_index("s") * NC + lax.axis_index("c")

    if pool:
        for cc in range(C // LANES):
            pool_v[pl.ds(cc * LANES, LANES)] = jnp.zeros((LANES,), jnp.float32)

    def chunk(ci, carry):
        c = wid * NCH + ci
        base = c * B
        pltpu.sync_copy(idx_hbm.at[c], idx_v)
        copies = [
            pltpu.async_copy(znf_hbm.at[idx_v.at[k]], gbuf_v.at[k], sem)
            for k in range(K)
        ]
        copies.append(
            pltpu.async_copy(zself_hbm.at[pl.ds(base, B)], acc_v, sem))
        for cp in copies:
            cp.wait()
        _sc_combine_rows(acc_v, gbuf_v, pool_v if pool else None)
        if not pool:
            pltpu.sync_copy(acc_v, out_hbm.at[pl.ds(base, B)])
        return carry

    lax.fori_loop(0, NCH, chunk, 0)

    if pool:
        pltpu.sync_copy(pool_v, out_hbm.at[wid])


@functools.lru_cache(maxsize=None)
def _make_sc_conv(pool):
    mesh = plsc.VectorSubcoreMesh(core_axis_name="c", subcore_axis_name="s",
                                  num_cores=NC, num_subcores=NS)
    out_shape = (NW, C) if pool else (N, C)
    scratch = [
        pltpu.VMEM((K, B), jnp.int32),
        pltpu.VMEM((B, C), jnp.float32),
        pltpu.VMEM((K, B, C), jnp.float32),
        pltpu.VMEM((C,), jnp.float32),
        pltpu.SemaphoreType.DMA,
    ]
    return pl.kernel(
        functools.partial(_sc_body, pool),
        out_type=jax.ShapeDtypeStruct(out_shape, jnp.float32),
        mesh=mesh,
        scratch_types=scratch,
    )


def _fc_body(part_ref, wfcw_ref, bfc_ref, out_ref):
    acc = bfc_ref[...]
    for w in range(NW):
        acc = acc + jnp.dot(part_ref[w:w + 1, :], wfcw_ref[w],
                            preferred_element_type=jnp.float32)
    out_ref[...] = acc


@functools.lru_cache(maxsize=None)
def _make_fc():
    return pl.pallas_call(
        _fc_body,
        in_specs=[
            pl.BlockSpec((NW, C), lambda: (0, 0)),
            pl.BlockSpec((NW, C, CAT), lambda: (0, 0, 0)),
            pl.BlockSpec((1, CAT), lambda: (0, 0)),
        ],
        out_specs=pl.BlockSpec((1, CAT), lambda: (0, 0)),
        out_shape=jax.ShapeDtypeStruct((1, CAT), jnp.float32),
    )


def _split_weights(W):
    Wr = W.reshape(C, K + 1, C)
    ws = Wr[:, 0, :].T
    wn = Wr[:, 1:, :].transpose(1, 2, 0)
    return ws, wn


def kernel(x, half_edges, W0, b0, W1, b1, W2, b2, Wfc, bfc):
    he = half_edges.astype(jnp.int32)
    idx_full = he.T + (jnp.arange(K, dtype=jnp.int32) * N)[:, None]
    idx_tiled = idx_full.reshape(K, NW * NCH, B).transpose(1, 0, 2)

    project = _make_project(N, 3200)
    sc_conv = _make_sc_conv(False)
    sc_pool = _make_sc_conv(True)

    h = x
    for (W, b), last in (((W0, b0), False), ((W1, b1), False), ((W2, b2), True)):
        ws, wn = _split_weights(W)
        zself, zn = project(h, ws, wn, b.reshape(1, C))
        znf = zn.reshape(K * N, C)
        if last:
            partials = sc_pool(zself, znf, idx_tiled)
        else:
            h = sc_conv(zself, znf, idx_tiled)

    wf3 = Wfc.reshape(CAT, P, C).transpose(1, 2, 0)
    wfcw = jnp.repeat(wf3, NW // P, axis=0) * (1.0 / (N // P))
    out = _make_fc()(partials, wfcw, bfc.reshape(1, CAT))
    return out.reshape(CAT)


# speedup vs baseline: 8.0698x; 1.1353x over previous
import functools

import jax
import jax.numpy as jnp
from jax import lax
from jax.experimental import pallas as pl
from jax.experimental.pallas import tpu as pltpu
from jax.experimental.pallas import tpu_sc as plsc

N = 320000
C = 128
K = 4
P = 16
CAT = 32

NC = 2
NS = 16
NW = NC * NS
ROWS_W = N // NW
B = 80
NCH = ROWS_W // B
LANES = 16


def _proj_body(x_ref, ws_ref, wn_ref, b_ref, zself_ref, zn_ref):
    xb = x_ref[...]
    zself_ref[...] = (
        jnp.dot(xb, ws_ref[...], preferred_element_type=jnp.float32)
        + b_ref[...]
    )
    for k in range(K):
        zn_ref[k] = jnp.dot(xb, wn_ref[k], preferred_element_type=jnp.float32)


@functools.lru_cache(maxsize=None)
def _make_project(n, bn):
    grid = n // bn
    return pl.pallas_call(
        _proj_body,
        grid=(grid,),
        in_specs=[
            pl.BlockSpec((bn, C), lambda i: (i, 0)),
            pl.BlockSpec((C, C), lambda i: (0, 0)),
            pl.BlockSpec((K, C, C), lambda i: (0, 0, 0)),
            pl.BlockSpec((1, C), lambda i: (0, 0)),
        ],
        out_specs=[
            pl.BlockSpec((bn, C), lambda i: (i, 0)),
            pl.BlockSpec((K, bn, C), lambda i: (0, i, 0)),
        ],
        out_shape=[
            jax.ShapeDtypeStruct((n, C), jnp.float32),
            jax.ShapeDtypeStruct((K, n, C), jnp.float32),
        ],
    )


def _sc_combine_rows(acc_v, gbuf_v, pool_v):
    def row(r, carry):
        for cc in range(C // LANES):
            sl = pl.ds(cc * LANES, LANES)
            v = acc_v[r, sl]
            for k in range(K):
                v = v + gbuf_v[k, r, sl]
            v = jnp.maximum(v, 0.0)
            if pool_v is None:
                acc_v[r, sl] = v
            else:
                pool_v[sl] = pool_v[sl] + v
        return carry

    lax.fori_loop(0, B, row, 0)


def _sc_body(pool, zself_hbm, znf_hbm, idx_hbm, out_hbm,
             idx_v, acc_v, gbuf_v, pool_v, sem):
    wid = lax.axis_index("s") * NC + lax.axis_index("c")

    if pool:
        for cc in range(C // LANES):
            pool_v[pl.ds(cc * LANES, LANES)] = jnp.zeros((LANES,), jnp.float32)

    def chunk(ci, carry):
        c = wid * NCH + ci
        base = c * B
        pltpu.sync_copy(idx_hbm.at[c], idx_v)
        copies = [
            pltpu.async_copy(znf_hbm.at[idx_v.at[k]], gbuf_v.at[k], sem)
            for k in range(K)
        ]
        copies.append(
            pltpu.async_copy(zself_hbm.at[pl.ds(base, B)], acc_v, sem))
        for cp in copies:
            cp.wait()
        _sc_combine_rows(acc_v, gbuf_v, pool_v if pool else None)
        if not pool:
            pltpu.sync_copy(acc_v, out_hbm.at[pl.ds(base, B)])
        return carry

    lax.fori_loop(0, NCH, chunk, 0)

    if pool:
        pltpu.sync_copy(pool_v, out_hbm.at[wid])


@functools.lru_cache(maxsize=None)
def _make_sc_conv(pool):
    mesh = plsc.VectorSubcoreMesh(core_axis_name="c", subcore_axis_name="s",
                                  num_cores=NC, num_subcores=NS)
    out_shape = (NW, C) if pool else (N, C)
    scratch = [
        pltpu.VMEM((K, B), jnp.int32),
        pltpu.VMEM((B, C), jnp.float32),
        pltpu.VMEM((K, B, C), jnp.float32),
        pltpu.VMEM((C,), jnp.float32),
        pltpu.SemaphoreType.DMA,
    ]
    return pl.kernel(
        functools.partial(_sc_body, pool),
        out_type=jax.ShapeDtypeStruct(out_shape, jnp.float32),
        mesh=mesh,
        scratch_types=scratch,
    )


BN_POOL = 4000
NBLK = N // BN_POOL
BLK_PER_BIN = (N // P) // BN_POOL


def _pool_fc_body(h_ref, wf3_ref, bfc_ref, out_ref, pooled_ref):
    i = pl.program_id(0)
    r = i // BLK_PER_BIN
    s = jnp.sum(h_ref[...], axis=0, keepdims=True)

    @pl.when(i % BLK_PER_BIN == 0)
    def _init():
        pooled_ref[pl.ds(r, 1), :] = s

    @pl.when(i % BLK_PER_BIN != 0)
    def _acc():
        pooled_ref[pl.ds(r, 1), :] = pooled_ref[pl.ds(r, 1), :] + s

    @pl.when(i == NBLK - 1)
    def _fc():
        acc = bfc_ref[...]
        for p in range(P):
            acc = acc + jnp.dot(pooled_ref[p:p + 1, :], wf3_ref[p],
                                preferred_element_type=jnp.float32)
        out_ref[...] = acc


@functools.lru_cache(maxsize=None)
def _make_pool_fc():
    return pl.pallas_call(
        _pool_fc_body,
        grid=(NBLK,),
        in_specs=[
            pl.BlockSpec((BN_POOL, C), lambda i: (i, 0)),
            pl.BlockSpec((P, C, CAT), lambda i: (0, 0, 0)),
            pl.BlockSpec((1, CAT), lambda i: (0, 0)),
        ],
        out_specs=pl.BlockSpec((1, CAT), lambda i: (0, 0)),
        out_shape=jax.ShapeDtypeStruct((1, CAT), jnp.float32),
        scratch_shapes=[pltpu.VMEM((P, C), jnp.float32)],
    )


def _split_weights(W):
    Wr = W.reshape(C, K + 1, C)
    ws = Wr[:, 0, :].T
    wn = Wr[:, 1:, :].transpose(1, 2, 0)
    return ws, wn


def kernel(x, half_edges, W0, b0, W1, b1, W2, b2, Wfc, bfc):
    he = half_edges.astype(jnp.int32)
    idx_full = he.T + (jnp.arange(K, dtype=jnp.int32) * N)[:, None]
    idx_tiled = idx_full.reshape(K, NW * NCH, B).transpose(1, 0, 2)

    project = _make_project(N, 3200)
    sc_conv = _make_sc_conv(False)

    h = x
    for W, b in ((W0, b0), (W1, b1), (W2, b2)):
        ws, wn = _split_weights(W)
        zself, zn = project(h, ws, wn, b.reshape(1, C))
        znf = zn.reshape(K * N, C)
        h = sc_conv(zself, znf, idx_tiled)

    wf3 = Wfc.reshape(CAT, P, C).transpose(1, 2, 0) * (1.0 / (N // P))
    out = _make_pool_fc()(h, wf3, bfc.reshape(1, CAT))
    return out.reshape(CAT)


# speedup vs baseline: 11.7539x; 1.4565x over previous
import functools

import jax
import jax.numpy as jnp
from jax import lax
from jax.experimental import pallas as pl
from jax.experimental.pallas import tpu as pltpu
from jax.experimental.pallas import tpu_sc as plsc

N = 320000
C = 128
K = 4
P = 16
CAT = 32

NC = 2
NS = 16
NW = NC * NS
ROWS_W = N // NW
B = 80
NCH = ROWS_W // B
LANES = 16


def _proj_body(x_ref, ws_ref, wn_ref, b_ref, zself_ref, zn_ref):
    xb = x_ref[...]
    zself_ref[...] = (
        jnp.dot(xb, ws_ref[...], preferred_element_type=jnp.float32)
        + b_ref[...]
    )
    for k in range(K):
        zn_ref[k] = jnp.dot(xb, wn_ref[k], preferred_element_type=jnp.float32)


@functools.lru_cache(maxsize=None)
def _make_project(n, bn):
    grid = n // bn
    return pl.pallas_call(
        _proj_body,
        grid=(grid,),
        in_specs=[
            pl.BlockSpec((bn, C), lambda i: (i, 0)),
            pl.BlockSpec((C, C), lambda i: (0, 0)),
            pl.BlockSpec((K, C, C), lambda i: (0, 0, 0)),
            pl.BlockSpec((1, C), lambda i: (0, 0)),
        ],
        out_specs=[
            pl.BlockSpec((bn, C), lambda i: (i, 0)),
            pl.BlockSpec((K, bn, C), lambda i: (0, i, 0)),
        ],
        out_shape=[
            jax.ShapeDtypeStruct((n, C), jnp.float32),
            jax.ShapeDtypeStruct((K, n, C), jnp.float32),
        ],
    )


def _sc_combine_rows(acc_v, gbuf_v):
    def row(r, carry):
        for cc in range(C // LANES):
            sl = pl.ds(cc * LANES, LANES)
            g01 = gbuf_v[0, r, sl] + gbuf_v[1, r, sl]
            g23 = gbuf_v[2, r, sl] + gbuf_v[3, r, sl]
            v = acc_v[r, sl] + (g01 + g23)
            acc_v[r, sl] = jnp.maximum(v, 0.0)
        return carry

    lax.fori_loop(0, B, row, 0)


def _sc_body(zself_hbm, znf_hbm, idx_hbm, out_hbm,
             idx0, idx1, acc0, acc1, gb0, gb1, sg0, sg1, si0, si1):
    wid = lax.axis_index("s") * NC + lax.axis_index("c")
    base_c = wid * NCH
    idxb, accb, gbb = (idx0, idx1), (acc0, acc1), (gb0, gb1)
    sgb, sib = (sg0, sg1), (si0, si1)

    def issue(ci, b):
        for k in range(K):
            pltpu.async_copy(znf_hbm.at[idxb[b].at[k]], gbb[b].at[k], sgb[b])
        pltpu.async_copy(
            zself_hbm.at[pl.ds((base_c + ci) * B, B)], accb[b], sgb[b])

    def drain(b):
        for k in range(K):
            pltpu.make_async_copy(
                znf_hbm.at[idxb[b].at[k]], gbb[b].at[k], sgb[b]).wait()
        pltpu.make_async_copy(
            zself_hbm.at[pl.ds(0, B)], accb[b], sgb[b]).wait()

    def issue_idx(ci, b):
        pltpu.async_copy(idx_hbm.at[base_c + ci], idxb[b], sib[b])

    def drain_idx(b):
        pltpu.make_async_copy(idx_hbm.at[0], idxb[b], sib[b]).wait()

    def process(ci, b):
        drain(b)

        @pl.when(ci + 1 < NCH)
        def _start_next():
            drain_idx(1 - b)
            issue(ci + 1, 1 - b)

        @pl.when(ci + 2 < NCH)
        def _prefetch_idx():
            issue_idx(ci + 2, b)

        _sc_combine_rows(accb[b], gbb[b])
        pltpu.sync_copy(accb[b], out_hbm.at[pl.ds((base_c + ci) * B, B)])

    pltpu.sync_copy(idx_hbm.at[base_c], idxb[0])
    issue(0, 0)
    issue_idx(1, 1)

    def pair(i, carry):
        ci = i * 2
        process(ci, 0)

        @pl.when(ci + 1 < NCH)
        def _odd():
            process(ci + 1, 1)

        return carry

    lax.fori_loop(0, (NCH + 1) // 2, pair, 0)


@functools.lru_cache(maxsize=None)
def _make_sc_conv():
    mesh = plsc.VectorSubcoreMesh(core_axis_name="c", subcore_axis_name="s",
                                  num_cores=NC, num_subcores=NS)
    scratch = [
        pltpu.VMEM((K, B), jnp.int32),
        pltpu.VMEM((K, B), jnp.int32),
        pltpu.VMEM((B, C), jnp.float32),
        pltpu.VMEM((B, C), jnp.float32),
        pltpu.VMEM((K, B, C), jnp.float32),
        pltpu.VMEM((K, B, C), jnp.float32),
        pltpu.SemaphoreType.DMA,
        pltpu.SemaphoreType.DMA,
        pltpu.SemaphoreType.DMA,
        pltpu.SemaphoreType.DMA,
    ]
    return pl.kernel(
        _sc_body,
        out_type=jax.ShapeDtypeStruct((N, C), jnp.float32),
        mesh=mesh,
        scratch_types=scratch,
    )


BN_POOL = 4000
NBLK = N // BN_POOL
BLK_PER_BIN = (N // P) // BN_POOL


def _pool_fc_body(h_ref, wf3_ref, bfc_ref, out_ref, pooled_ref):
    i = pl.program_id(0)
    r = i // BLK_PER_BIN
    s = jnp.sum(h_ref[...], axis=0, keepdims=True)

    @pl.when(i % BLK_PER_BIN == 0)
    def _init():
        pooled_ref[pl.ds(r, 1), :] = s

    @pl.when(i % BLK_PER_BIN != 0)
    def _acc():
        pooled_ref[pl.ds(r, 1), :] = pooled_ref[pl.ds(r, 1), :] + s

    @pl.when(i == NBLK - 1)
    def _fc():
        acc = bfc_ref[...]
        for p in range(P):
            acc = acc + jnp.dot(pooled_ref[p:p + 1, :], wf3_ref[p],
                                preferred_element_type=jnp.float32)
        out_ref[...] = acc


@functools.lru_cache(maxsize=None)
def _make_pool_fc():
    return pl.pallas_call(
        _pool_fc_body,
        grid=(NBLK,),
        in_specs=[
            pl.BlockSpec((BN_POOL, C), lambda i: (i, 0)),
            pl.BlockSpec((P, C, CAT), lambda i: (0, 0, 0)),
            pl.BlockSpec((1, CAT), lambda i: (0, 0)),
        ],
        out_specs=pl.BlockSpec((1, CAT), lambda i: (0, 0)),
        out_shape=jax.ShapeDtypeStruct((1, CAT), jnp.float32),
        scratch_shapes=[pltpu.VMEM((P, C), jnp.float32)],
    )


def _split_weights(W):
    Wr = W.reshape(C, K + 1, C)
    ws = Wr[:, 0, :].T
    wn = Wr[:, 1:, :].transpose(1, 2, 0)
    return ws, wn


def kernel(x, half_edges, W0, b0, W1, b1, W2, b2, Wfc, bfc):
    he = half_edges.astype(jnp.int32)
    idx_full = he.T + (jnp.arange(K, dtype=jnp.int32) * N)[:, None]
    idx_tiled = idx_full.reshape(K, NW * NCH, B).transpose(1, 0, 2)

    project = _make_project(N, 3200)
    sc_conv = _make_sc_conv()

    h = x
    for W, b in ((W0, b0), (W1, b1), (W2, b2)):
        ws, wn = _split_weights(W)
        zself, zn = project(h, ws, wn, b.reshape(1, C))
        znf = zn.reshape(K * N, C)
        h = sc_conv(zself, znf, idx_tiled)

    wf3 = Wfc.reshape(CAT, P, C).transpose(1, 2, 0) * (1.0 / (N // P))
    out = _make_pool_fc()(h, wf3, bfc.reshape(1, CAT))
    return out.reshape(CAT)
